# baseline (speedup 1.0000x reference)
import functools

import jax
import jax.numpy as jnp
from jax import lax
from jax.experimental import pallas as pl
from jax.experimental.pallas import tpu as pltpu
from jax.experimental.pallas import tpu_sc as plsc

_NUM_WORKERS = 32
_CHUNK = 512


@functools.lru_cache(maxsize=None)
def _make_gather(V, D, B):
    C = _CHUNK
    b_per_w = B // _NUM_WORKERS
    n_chunks = b_per_w // C
    n_pairs = n_chunks // 2
    mesh = plsc.VectorSubcoreMesh(core_axis_name="c", subcore_axis_name="s")

    @functools.partial(
        pl.kernel,
        mesh=mesh,
        compiler_params=pltpu.CompilerParams(use_tc_tiling_on_sc=False),
        out_type=jax.ShapeDtypeStruct((B, D), jnp.float32),
        scratch_types=[
            pltpu.VMEM((b_per_w,), jnp.int32),
            pltpu.VMEM((C, D), jnp.float32),
            pltpu.VMEM((C, D), jnp.float32),
            pltpu.SemaphoreType.DMA,
            pltpu.SemaphoreType.DMA,
        ],
    )
    def gather_kernel(idx_hbm, table_hbm, out_hbm, idx_v, r0, r1, gsem, wsem):
        wid = lax.axis_index("s") * 2 + lax.axis_index("c")
        base = wid * b_per_w
        pltpu.sync_copy(idx_hbm.at[pl.ds(base, b_per_w)], idx_v)

        def fire_gather(g, buf):
            pltpu.async_copy(table_hbm.at[idx_v.at[pl.ds(g * C, C)]], buf, gsem)

        def wait_gather(buf):
            pltpu.make_async_copy(table_hbm.at[pl.ds(0, C)], buf, gsem).wait()

        def fire_write(g, buf):
            pltpu.async_copy(buf, out_hbm.at[pl.ds(base + g * C, C)], wsem)

        def wait_write(buf):
            pltpu.make_async_copy(buf, out_hbm.at[pl.ds(base, C)], wsem).wait()

        fire_gather(0, r0)
        fire_gather(1, r1)
        wait_gather(r0)
        fire_write(0, r0)
        wait_gather(r1)
        fire_write(1, r1)

        def body(i, carry):
            g0 = 2 * i
            wait_write(r0)
            fire_gather(g0, r0)
            wait_gather(r0)
            fire_write(g0, r0)
            wait_write(r1)
            fire_gather(g0 + 1, r1)
            wait_gather(r1)
            fire_write(g0 + 1, r1)
            return carry

        lax.fori_loop(1, n_pairs, body, 0)
        wait_write(r0)
        wait_write(r1)

    return gather_kernel


def kernel(input, W):
    Bm, F = input.shape
    V, D = W.shape
    B = Bm * F
    idx_flat = input.reshape(B)
    out = _make_gather(V, D, B)(idx_flat, W)
    return out.reshape(Bm, F, D)
